# baseline (speedup 1.0000x reference)
import jax
import jax.numpy as jnp
from jax import lax
from jax.experimental import pallas as pl
from jax.experimental.pallas import tpu as pltpu
from jax.experimental.pallas import tpu_sc as plsc

_B = 1024
_N = 100000
_E = 128
_H = 4
_K = 6
_HD = _E // _H
_V = 128
_NBLK_PAD = 896
_NV = _NBLK_PAD // 16
_W = 16384
_NT = 7
_RTA = 256
_RT = 256
_TAIL = 781
_TS = _TAIL * _V - 6 * _W
_NC = 2
_NS = 16
_RPW = _B // (_NC * _NS)
_Q = 256
_NEG = float("-inf")


def _blockmax_body(x_ref, bm_ref, aux_ref):
    j = pl.program_id(1)

    @pl.when(j < _NT - 1)
    def _():
        x = x_ref[...]
        bm_ref[...] = jnp.max(x.reshape(_RTA, _W // _V, _V), axis=2)

    @pl.when(j == _NT - 1)
    def _():
        x = x_ref[...]
        col = (_NT - 1) * _W + lax.broadcasted_iota(jnp.int32, (_RTA, _W), 1)
        x = jnp.where(col < _N, x, _NEG)
        bm_ref[...] = jnp.max(x.reshape(_RTA, _W // _V, _V), axis=2)
        aux_ref[...] = x[:, _TS:_TS + _V]


def _blockmax(logits, h):
    return pl.pallas_call(
        _blockmax_body,
        grid=(1, _NT),
        in_specs=[pl.BlockSpec((_RTA, _W), lambda i, j: (h + i, j))],
        out_specs=[
            pl.BlockSpec((_RTA, _W // _V), lambda i, j: (i, j)),
            pl.BlockSpec((_RTA, _V), lambda i, j: (i, 0)),
        ],
        out_shape=[
            jax.ShapeDtypeStruct((_Q, _NBLK_PAD), jnp.float32),
            jax.ShapeDtypeStruct((_Q, _V), jnp.float32),
        ],
        compiler_params=pltpu.CompilerParams(
            dimension_semantics=("parallel", "arbitrary")),
    )(logits)


def _blocksel_body(bm_ref, ids_ref):
    x = bm_ref[...]
    ii = lax.broadcasted_iota(jnp.int32, x.shape, 1)
    cols = []
    for _ in range(_K):
        m = jnp.max(x, axis=1, keepdims=True)
        idk = jnp.min(jnp.where(x == m, ii, jnp.int32(2 * _NBLK_PAD)),
                      axis=1, keepdims=True)
        x = jnp.where(ii == idk, _NEG, x)
        cols.append(idk)
    pad = jnp.zeros((x.shape[0], 16 - _K), jnp.int32)
    ids_ref[...] = jnp.concatenate(cols + [pad], axis=1)


def _blocksel(bm):
    return pl.pallas_call(
        _blocksel_body,
        grid=(_Q // _RT,),
        in_specs=[pl.BlockSpec((_RT, _NBLK_PAD), lambda i: (i, 0))],
        out_specs=pl.BlockSpec((_RT, 16), lambda i: (i, 0)),
        out_shape=jax.ShapeDtypeStruct((_Q, 16), jnp.int32),
    )(bm)


def _mesh():
    return plsc.VectorSubcoreMesh(core_axis_name="c", subcore_axis_name="s",
                                  num_cores=_NC, num_subcores=_NS)


def _b1_body(h, ids_hbm, logits_hbm, aux_hbm, cand_hbm, ids_s, cand_s, crow_s,
             *sems):
    wid = lax.axis_index("s") * _NC + lax.axis_index("c")

    if True:
        rbase = pl.multiple_of(wid * 8, 8)
        rbase_g = pl.multiple_of(h * _Q + wid * 8, 8)
        pltpu.sync_copy(ids_hbm.at[pl.ds(rbase, 8)], ids_s)

        for rm in range(8):
            idv = ids_s[rm]
            for kk in range(_K):
                blk = jnp.minimum(idv[kk], _TAIL - 1)
                start = pl.multiple_of(blk * _V, _V)
                slot = rm * _K + kk
                pltpu.async_copy(
                    logits_hbm.at[pl.ds(rbase_g, 8), pl.ds(start, _V)],
                    cand_s.at[pl.ds(slot * 8, 8)], sems[0])
        pltpu.make_async_copy(aux_hbm.at[pl.ds(0, 8 * _K * 8)], cand_s,
                              sems[0]).wait()

        def row_body(rm, carry):
            for kk in range(_K):
                slot = rm * _K + kk
                for iv in range(_V // 16):
                    crow_s[rm, pl.ds(kk * _V + iv * 16, 16)] = \
                        cand_s[slot * 8 + rm, pl.ds(iv * 16, 16)]
            return carry

        lax.fori_loop(0, 8, row_body, jnp.int32(0))
        pltpu.sync_copy(crow_s, cand_hbm.at[pl.ds(rbase, 8)])


def _sc_gather_cand(ids, aux, logits, h):
    import functools
    f = pl.kernel(
        functools.partial(_b1_body, h),
        out_type=jax.ShapeDtypeStruct((_Q, _K * _V), jnp.float32),
        mesh=_mesh(),
        scratch_types=[
            pltpu.VMEM((8, 16), jnp.int32),
            pltpu.VMEM((8 * _K * 8, _V), jnp.float32),
            pltpu.VMEM((8, _K * _V), jnp.float32),
            pltpu.SemaphoreType.DMA,
        ],
    )
    return f(ids, logits, aux)


def _extract_body(cand_ref, ids_ref, aux_ref, idx_ref):
    idv = ids_ref[...]
    aux = aux_ref[...]
    t = lax.broadcasted_iota(jnp.int32, (_RT, _V), 1)
    pieces, gidxs = [], []
    for kk in range(_K):
        idk = idv[:, kk:kk + 1]
        ck = cand_ref[:, kk * _V:(kk + 1) * _V]
        pieces.append(jnp.where(idk == _TAIL, aux, ck))
        gidxs.append(idk * _V + t)
    cand = jnp.concatenate(pieces, axis=1)
    gidx = jnp.concatenate(gidxs, axis=1)
    outs = []
    for _ in range(_K):
        m = jnp.max(cand, axis=1, keepdims=True)
        sel = jnp.min(jnp.where(cand == m, gidx, jnp.int32(2 * _N)),
                      axis=1, keepdims=True)
        cand = jnp.where(gidx == sel, _NEG, cand)
        outs.append(sel)
    pad = jnp.zeros((_RT, 16 - _K), jnp.int32)
    idx_ref[...] = jnp.concatenate(outs + [pad], axis=1)


def _extract(cand, ids, aux):
    return pl.pallas_call(
        _extract_body,
        grid=(_Q // _RT,),
        in_specs=[pl.BlockSpec((_RT, _K * _V), lambda i: (i, 0)),
                  pl.BlockSpec((_RT, 16), lambda i: (i, 0)),
                  pl.BlockSpec((_RT, _V), lambda i: (i, 0))],
        out_specs=pl.BlockSpec((_RT, 16), lambda i: (i, 0)),
        out_shape=jax.ShapeDtypeStruct((_Q, 16), jnp.int32),
    )(cand, ids, aux)


def _b2_body(idx_hbm, bank_hbm, sel_hbm, idx_s, rows8_v, sem_b):
    wid = lax.axis_index("s") * _NC + lax.axis_index("c")

    if True:
        rbase = pl.multiple_of(wid * 8, 8)
        pltpu.sync_copy(idx_hbm.at[pl.ds(rbase, 8)], idx_s)
        for rm in range(8):
            pltpu.async_copy(bank_hbm.at[idx_s.at[rm]], rows8_v.at[rm], sem_b)
        for rm in range(8):
            pltpu.make_async_copy(bank_hbm.at[pl.ds(0, 16)],
                                  rows8_v.at[rm], sem_b).wait()
        pltpu.sync_copy(rows8_v.at[:, pl.ds(0, 8)],
                        sel_hbm.at[pl.ds(rbase, 8)])


def _sc_bank_gather(idx, bank):
    f = pl.kernel(
        _b2_body,
        out_type=jax.ShapeDtypeStruct((_Q, 8, _E), jnp.float32),
        mesh=_mesh(),
        scratch_types=[
            pltpu.VMEM((8, 16), jnp.int32),
            pltpu.VMEM((8, 16, _E), jnp.float32),
            pltpu.SemaphoreType.DMA,
        ],
    )
    return f(idx, bank)


def _attn_body(q_ref, sel_ref, wq_ref, bq_ref, wp_ref, bp_ref,
               inw_ref, inb_ref, outw_ref, outb_ref, ctx_ref, aw_ref):
    f32 = jnp.float32

    def dot_t(a, b):
        return lax.dot_general(a, b, (((1,), (1,)), ((), ())),
                               preferred_element_type=f32)

    q = q_ref[...]
    aq = dot_t(q, wq_ref[...]) + bq_ref[...]
    qp = dot_t(aq, inw_ref[0:_E, :]) + inb_ref[0:1, :]

    d_i = lax.broadcasted_iota(jnp.int32, (_E, _H), 0)
    h_i = lax.broadcasted_iota(jnp.int32, (_E, _H), 1)
    sel_m = (d_i // _HD == h_i).astype(f32)
    scale = _HD ** -0.5

    ts, vs = [], []
    for j in range(_K):
        kv = dot_t(sel_ref[:, j, :], wp_ref[...]) + bp_ref[...]
        kp = dot_t(kv, inw_ref[_E:2 * _E, :]) + inb_ref[1:2, :]
        vp = dot_t(kv, inw_ref[2 * _E:3 * _E, :]) + inb_ref[2:3, :]
        t = lax.dot_general(qp * kp, sel_m, (((1,), (0,)), ((), ())),
                            preferred_element_type=f32) * scale
        ts.append(t)
        vs.append(vp)

    m = ts[0]
    for t in ts[1:]:
        m = jnp.maximum(m, t)
    es = [jnp.exp(t - m) for t in ts]
    z = es[0]
    for e in es[1:]:
        z = z + e
    ws = [e / z for e in es]

    aw = jnp.concatenate(
        [jnp.sum(w, axis=1, keepdims=True) for w in ws], axis=1) * (1.0 / _H)

    ctx = jnp.zeros_like(qp)
    for j in range(_K):
        wexp = dot_t(ws[j], sel_m)
        ctx = ctx + wexp * vs[j]
    ctx_ref[...] = dot_t(ctx, outw_ref[...]) + outb_ref[...]
    aw_ref[...] = aw


def _attn(query, sel, h, W_q, b_q, W_p, b_p, inw, inb, outw, outb):
    def full(shape):
        return pl.BlockSpec(shape, lambda i: tuple(0 for _ in shape))
    return pl.pallas_call(
        _attn_body,
        grid=(_Q // _RT,),
        in_specs=[
            pl.BlockSpec((_RT, _E), lambda i: (h + i, 0)),
            pl.BlockSpec((_RT, 8, _E), lambda i: (i, 0, 0)),
            full((_E, _E)), full((1, _E)),
            full((_E, _E)), full((1, _E)),
            full((3 * _E, _E)), full((3, _E)),
            full((_E, _E)), full((1, _E)),
        ],
        out_specs=[
            pl.BlockSpec((_RT, _E), lambda i: (i, 0)),
            pl.BlockSpec((_RT, _K), lambda i: (i, 0)),
        ],
        out_shape=[
            jax.ShapeDtypeStruct((_Q, _E), jnp.float32),
            jax.ShapeDtypeStruct((_Q, _K), jnp.float32),
        ],
    )(query, sel, W_q, b_q.reshape(1, _E), W_p, b_p.reshape(1, _E),
      inw, inb.reshape(3, _E), outw, outb.reshape(1, _E))


def kernel(query, prototype_bank, prototype_logits, W_q_proj, b_q_proj,
           W_p_proj, b_p_proj, in_proj_w, in_proj_b, out_proj_w, out_proj_b):
    ctxs, aws = [], []
    for h in range(_B // _Q):
        bm, aux = _blockmax(prototype_logits, h)
        ids = _blocksel(bm)
        cand = _sc_gather_cand(ids, aux, prototype_logits, h)
        idx = _extract(cand, ids, aux)
        sel = _sc_bank_gather(idx, prototype_bank)
        ctx, aw = _attn(query, sel, h, W_q_proj, b_q_proj, W_p_proj, b_p_proj,
                        in_proj_w, in_proj_b, out_proj_w, out_proj_b)
        ctxs.append(ctx)
        aws.append(aw)
    return jnp.concatenate(ctxs, axis=0), jnp.concatenate(aws, axis=0)


# speedup vs baseline: 1.0168x; 1.0168x over previous
import jax
import jax.numpy as jnp
from jax import lax
from jax.experimental import pallas as pl
from jax.experimental.pallas import tpu as pltpu
from jax.experimental.pallas import tpu_sc as plsc

_B = 1024
_N = 100000
_E = 128
_H = 4
_K = 6
_HD = _E // _H
_V = 128
_NBLK_PAD = 896
_NV = _NBLK_PAD // 16
_W = 16384
_NT = 7
_RTA = 256
_RT = 256
_TAIL = 781
_TS = _TAIL * _V - 6 * _W
_NC = 2
_NS = 16
_RPW = _B // (_NC * _NS)
_Q = 1024
_NEG = float("-inf")


def _blockmax_body(x_ref, bm_ref, aux_ref):
    j = pl.program_id(1)

    @pl.when(j < _NT - 1)
    def _():
        x = x_ref[...]
        bm_ref[...] = jnp.max(x.reshape(_RTA, _W // _V, _V), axis=2)

    @pl.when(j == _NT - 1)
    def _():
        x = x_ref[...]
        col = (_NT - 1) * _W + lax.broadcasted_iota(jnp.int32, (_RTA, _W), 1)
        x = jnp.where(col < _N, x, _NEG)
        bm_ref[...] = jnp.max(x.reshape(_RTA, _W // _V, _V), axis=2)
        aux_ref[...] = x[:, _TS:_TS + _V]


def _blockmax(logits, h):
    return pl.pallas_call(
        _blockmax_body,
        grid=(_Q // _RTA, _NT),
        in_specs=[pl.BlockSpec((_RTA, _W),
                               lambda i, j: (h * (_Q // _RTA) + i, j))],
        out_specs=[
            pl.BlockSpec((_RTA, _W // _V), lambda i, j: (i, j)),
            pl.BlockSpec((_RTA, _V), lambda i, j: (i, 0)),
        ],
        out_shape=[
            jax.ShapeDtypeStruct((_Q, _NBLK_PAD), jnp.float32),
            jax.ShapeDtypeStruct((_Q, _V), jnp.float32),
        ],
        compiler_params=pltpu.CompilerParams(
            dimension_semantics=("parallel", "arbitrary")),
    )(logits)


def _blocksel_body(bm_ref, ids_ref):
    x = bm_ref[...]
    ii = lax.broadcasted_iota(jnp.int32, x.shape, 1)
    cols = []
    for _ in range(_K):
        m = jnp.max(x, axis=1, keepdims=True)
        idk = jnp.min(jnp.where(x == m, ii, jnp.int32(2 * _NBLK_PAD)),
                      axis=1, keepdims=True)
        x = jnp.where(ii == idk, _NEG, x)
        cols.append(idk)
    pad = jnp.zeros((x.shape[0], 16 - _K), jnp.int32)
    ids_ref[...] = jnp.concatenate(cols + [pad], axis=1)


def _blocksel(bm):
    return pl.pallas_call(
        _blocksel_body,
        grid=(_Q // _RT,),
        in_specs=[pl.BlockSpec((_RT, _NBLK_PAD), lambda i: (i, 0))],
        out_specs=pl.BlockSpec((_RT, 16), lambda i: (i, 0)),
        out_shape=jax.ShapeDtypeStruct((_Q, 16), jnp.int32),
    )(bm)


def _mesh():
    return plsc.VectorSubcoreMesh(core_axis_name="c", subcore_axis_name="s",
                                  num_cores=_NC, num_subcores=_NS)


def _b1_body(h, ids_hbm, logits_hbm, aux_hbm, cand_hbm, ids_s, cand_s, crow_s,
             *sems):
    wid = lax.axis_index("s") * _NC + lax.axis_index("c")

    def slab_body(sb, carry0):
        rbase = pl.multiple_of(wid * _RPW + sb * 8, 8)
        rbase_g = pl.multiple_of(h * _Q + wid * _RPW + sb * 8, 8)
        pltpu.sync_copy(ids_hbm.at[pl.ds(rbase, 8)], ids_s)

        for rm in range(8):
            idv = ids_s[rm]
            for kk in range(_K):
                blk = jnp.minimum(idv[kk], _TAIL - 1)
                start = pl.multiple_of(blk * _V, _V)
                slot = rm * _K + kk
                pltpu.async_copy(
                    logits_hbm.at[pl.ds(rbase_g, 8), pl.ds(start, _V)],
                    cand_s.at[pl.ds(slot * 8, 8)], sems[0])
        pltpu.make_async_copy(aux_hbm.at[pl.ds(0, 8 * _K * 8)], cand_s,
                              sems[0]).wait()

        def row_body(rm, carry):
            for kk in range(_K):
                slot = rm * _K + kk
                for iv in range(_V // 16):
                    crow_s[rm, pl.ds(kk * _V + iv * 16, 16)] = \
                        cand_s[slot * 8 + rm, pl.ds(iv * 16, 16)]
            return carry

        lax.fori_loop(0, 8, row_body, jnp.int32(0))
        pltpu.sync_copy(crow_s, cand_hbm.at[pl.ds(rbase, 8)])
        return carry0

    lax.fori_loop(0, _RPW // 8, slab_body, jnp.int32(0))


def _sc_gather_cand(ids, aux, logits, h):
    import functools
    f = pl.kernel(
        functools.partial(_b1_body, h),
        out_type=jax.ShapeDtypeStruct((_Q, _K * _V), jnp.float32),
        mesh=_mesh(),
        scratch_types=[
            pltpu.VMEM((8, 16), jnp.int32),
            pltpu.VMEM((8 * _K * 8, _V), jnp.float32),
            pltpu.VMEM((8, _K * _V), jnp.float32),
            pltpu.SemaphoreType.DMA,
        ],
    )
    return f(ids, logits, aux)


def _extract_body(cand_ref, ids_ref, aux_ref, idx_ref):
    idv = ids_ref[...]
    aux = aux_ref[...]
    t = lax.broadcasted_iota(jnp.int32, (_RT, _V), 1)
    pieces, gidxs = [], []
    for kk in range(_K):
        idk = idv[:, kk:kk + 1]
        ck = cand_ref[:, kk * _V:(kk + 1) * _V]
        pieces.append(jnp.where(idk == _TAIL, aux, ck))
        gidxs.append(idk * _V + t)
    cand = jnp.concatenate(pieces, axis=1)
    gidx = jnp.concatenate(gidxs, axis=1)
    outs = []
    for _ in range(_K):
        m = jnp.max(cand, axis=1, keepdims=True)
        sel = jnp.min(jnp.where(cand == m, gidx, jnp.int32(2 * _N)),
                      axis=1, keepdims=True)
        cand = jnp.where(gidx == sel, _NEG, cand)
        outs.append(sel)
    pad = jnp.zeros((_RT, 16 - _K), jnp.int32)
    idx_ref[...] = jnp.concatenate(outs + [pad], axis=1)


def _extract(cand, ids, aux):
    return pl.pallas_call(
        _extract_body,
        grid=(_Q // _RT,),
        in_specs=[pl.BlockSpec((_RT, _K * _V), lambda i: (i, 0)),
                  pl.BlockSpec((_RT, 16), lambda i: (i, 0)),
                  pl.BlockSpec((_RT, _V), lambda i: (i, 0))],
        out_specs=pl.BlockSpec((_RT, 16), lambda i: (i, 0)),
        out_shape=jax.ShapeDtypeStruct((_Q, 16), jnp.int32),
    )(cand, ids, aux)


def _b2_body(idx_hbm, bank_hbm, sel_hbm, idx_s, rows8_v, sem_b):
    wid = lax.axis_index("s") * _NC + lax.axis_index("c")

    def slab_body(sb, carry0):
        rbase = pl.multiple_of(wid * _RPW + sb * 8, 8)
        pltpu.sync_copy(idx_hbm.at[pl.ds(rbase, 8)], idx_s)
        for rm in range(8):
            pltpu.async_copy(bank_hbm.at[idx_s.at[rm]], rows8_v.at[rm], sem_b)
        for rm in range(8):
            pltpu.make_async_copy(bank_hbm.at[pl.ds(0, 16)],
                                  rows8_v.at[rm], sem_b).wait()
        pltpu.sync_copy(rows8_v.at[:, pl.ds(0, 8)],
                        sel_hbm.at[pl.ds(rbase, 8)])
        return carry0

    lax.fori_loop(0, _RPW // 8, slab_body, jnp.int32(0))


def _sc_bank_gather(idx, bank):
    f = pl.kernel(
        _b2_body,
        out_type=jax.ShapeDtypeStruct((_Q, 8, _E), jnp.float32),
        mesh=_mesh(),
        scratch_types=[
            pltpu.VMEM((8, 16), jnp.int32),
            pltpu.VMEM((8, 16, _E), jnp.float32),
            pltpu.SemaphoreType.DMA,
        ],
    )
    return f(idx, bank)


def _attn_body(q_ref, sel_ref, wq_ref, bq_ref, wp_ref, bp_ref,
               inw_ref, inb_ref, outw_ref, outb_ref, ctx_ref, aw_ref):
    f32 = jnp.float32

    def dot_t(a, b):
        return lax.dot_general(a, b, (((1,), (1,)), ((), ())),
                               preferred_element_type=f32)

    q = q_ref[...]
    aq = dot_t(q, wq_ref[...]) + bq_ref[...]
    qp = dot_t(aq, inw_ref[0:_E, :]) + inb_ref[0:1, :]

    d_i = lax.broadcasted_iota(jnp.int32, (_E, _H), 0)
    h_i = lax.broadcasted_iota(jnp.int32, (_E, _H), 1)
    sel_m = (d_i // _HD == h_i).astype(f32)
    scale = _HD ** -0.5

    ts, vs = [], []
    for j in range(_K):
        kv = dot_t(sel_ref[:, j, :], wp_ref[...]) + bp_ref[...]
        kp = dot_t(kv, inw_ref[_E:2 * _E, :]) + inb_ref[1:2, :]
        vp = dot_t(kv, inw_ref[2 * _E:3 * _E, :]) + inb_ref[2:3, :]
        t = lax.dot_general(qp * kp, sel_m, (((1,), (0,)), ((), ())),
                            preferred_element_type=f32) * scale
        ts.append(t)
        vs.append(vp)

    m = ts[0]
    for t in ts[1:]:
        m = jnp.maximum(m, t)
    es = [jnp.exp(t - m) for t in ts]
    z = es[0]
    for e in es[1:]:
        z = z + e
    ws = [e / z for e in es]

    aw = jnp.concatenate(
        [jnp.sum(w, axis=1, keepdims=True) for w in ws], axis=1) * (1.0 / _H)

    ctx = jnp.zeros_like(qp)
    for j in range(_K):
        wexp = dot_t(ws[j], sel_m)
        ctx = ctx + wexp * vs[j]
    ctx_ref[...] = dot_t(ctx, outw_ref[...]) + outb_ref[...]
    aw_ref[...] = aw


def _attn(query, sel, h, W_q, b_q, W_p, b_p, inw, inb, outw, outb):
    def full(shape):
        return pl.BlockSpec(shape, lambda i: tuple(0 for _ in shape))
    return pl.pallas_call(
        _attn_body,
        grid=(_Q // _RT,),
        in_specs=[
            pl.BlockSpec((_RT, _E), lambda i: (h * (_Q // _RT) + i, 0)),
            pl.BlockSpec((_RT, 8, _E), lambda i: (i, 0, 0)),
            full((_E, _E)), full((1, _E)),
            full((_E, _E)), full((1, _E)),
            full((3 * _E, _E)), full((3, _E)),
            full((_E, _E)), full((1, _E)),
        ],
        out_specs=[
            pl.BlockSpec((_RT, _E), lambda i: (i, 0)),
            pl.BlockSpec((_RT, _K), lambda i: (i, 0)),
        ],
        out_shape=[
            jax.ShapeDtypeStruct((_Q, _E), jnp.float32),
            jax.ShapeDtypeStruct((_Q, _K), jnp.float32),
        ],
    )(query, sel, W_q, b_q.reshape(1, _E), W_p, b_p.reshape(1, _E),
      inw, inb.reshape(3, _E), outw, outb.reshape(1, _E))


def kernel(query, prototype_bank, prototype_logits, W_q_proj, b_q_proj,
           W_p_proj, b_p_proj, in_proj_w, in_proj_b, out_proj_w, out_proj_b):
    ctxs, aws = [], []
    for h in range(_B // _Q):
        bm, aux = _blockmax(prototype_logits, h)
        ids = _blocksel(bm)
        cand = _sc_gather_cand(ids, aux, prototype_logits, h)
        idx = _extract(cand, ids, aux)
        sel = _sc_bank_gather(idx, prototype_bank)
        ctx, aw = _attn(query, sel, h, W_q_proj, b_q_proj, W_p_proj, b_p_proj,
                        in_proj_w, in_proj_b, out_proj_w, out_proj_b)
        ctxs.append(ctx)
        aws.append(aw)
    return jnp.concatenate(ctxs, axis=0), jnp.concatenate(aws, axis=0)
